# Initial kernel scaffold; baseline (speedup 1.0000x reference)
import jax
import jax.numpy as jnp
from jax.experimental import pallas as pl


def kernel(x, edge_index, W1, a_src1, a_dst1, b1, W2, a_src2, a_dst2, b2):
    raise NotImplementedError("write your pallas kernel here")


# speedup vs baseline: 19.8298x; 19.8298x over previous
import functools

import jax
import jax.numpy as jnp
from jax import lax
from jax.experimental import pallas as pl
from jax.experimental.pallas import tpu as pltpu
from jax.experimental.pallas import tpu_sc as plsc

N = 10000
E = 320000
D = 128
NP = 10240
NPT = NP // 16
NB = 128
NW = 32
CH = 128
NCHUNK = 79
ET = CH * NCHUNK
EP = ET * NW
BM = 1024
BN = 1000
EPS = 1e-16


def _mm1_body(x_ref, w_ref, av_ref, bv_ref, h_ref, as_ref, ad_ref):
    h = jnp.dot(x_ref[...], w_ref[...], preferred_element_type=jnp.float32)
    h_ref[...] = h
    as_ref[...] = jnp.dot(h, av_ref[...])
    ad_ref[...] = jnp.dot(h, bv_ref[...])


def _mm2_body(raw_ref, den_ref, b_ref, w_ref, av_ref, bv_ref,
              h_ref, as_ref, ad_ref):
    raw = raw_ref[0] + raw_ref[1]
    den = den_ref[0] + den_ref[1]
    xin = jnp.maximum(raw / (den + EPS) + b_ref[...], 0.0)
    h = jnp.dot(xin, w_ref[...], preferred_element_type=jnp.float32)
    h_ref[...] = h
    as_ref[...] = jnp.dot(h, av_ref[...])
    ad_ref[...] = jnp.dot(h, bv_ref[...])


def _final_body(raw_ref, den_ref, b_ref, out_ref):
    raw = raw_ref[0] + raw_ref[1]
    den = den_ref[0] + den_ref[1]
    out_ref[...] = raw / (den + EPS) + b_ref[...]


def _dense1(xp, W, av, bv):
    return pl.pallas_call(
        _mm1_body,
        grid=(NP // BM,),
        in_specs=[
            pl.BlockSpec((BM, D), lambda i: (i, 0)),
            pl.BlockSpec((D, D), lambda i: (0, 0)),
            pl.BlockSpec((D, 1), lambda i: (0, 0)),
            pl.BlockSpec((D, 1), lambda i: (0, 0)),
        ],
        out_specs=[
            pl.BlockSpec((BM, D), lambda i: (i, 0)),
            pl.BlockSpec((BM, 1), lambda i: (i, 0)),
            pl.BlockSpec((BM, 1), lambda i: (i, 0)),
        ],
        out_shape=[
            jax.ShapeDtypeStruct((NP, D), jnp.float32),
            jax.ShapeDtypeStruct((NP, 1), jnp.float32),
            jax.ShapeDtypeStruct((NP, 1), jnp.float32),
        ],
    )(xp, W, av, bv)


def _dense2(raw, den, b, W, av, bv):
    return pl.pallas_call(
        _mm2_body,
        grid=(NP // BM,),
        in_specs=[
            pl.BlockSpec((2, BM, D), lambda i: (0, i, 0)),
            pl.BlockSpec((2, BM, 1), lambda i: (0, i, 0)),
            pl.BlockSpec((1, D), lambda i: (0, 0)),
            pl.BlockSpec((D, D), lambda i: (0, 0)),
            pl.BlockSpec((D, 1), lambda i: (0, 0)),
            pl.BlockSpec((D, 1), lambda i: (0, 0)),
        ],
        out_specs=[
            pl.BlockSpec((BM, D), lambda i: (i, 0)),
            pl.BlockSpec((BM, 1), lambda i: (i, 0)),
            pl.BlockSpec((BM, 1), lambda i: (i, 0)),
        ],
        out_shape=[
            jax.ShapeDtypeStruct((NP, D), jnp.float32),
            jax.ShapeDtypeStruct((NP, 1), jnp.float32),
            jax.ShapeDtypeStruct((NP, 1), jnp.float32),
        ],
    )(raw, den, b, W, av, bv)


def _finalize(raw, den, b):
    return pl.pallas_call(
        _final_body,
        grid=(N // BN,),
        in_specs=[
            pl.BlockSpec((2, BN, D), lambda i: (0, i, 0)),
            pl.BlockSpec((2, BN, 1), lambda i: (0, i, 0)),
            pl.BlockSpec((1, D), lambda i: (0, 0)),
        ],
        out_specs=pl.BlockSpec((BN, D), lambda i: (i, 0)),
        out_shape=jax.ShapeDtypeStruct((N, D), jnp.float32),
    )(raw, den, b)


@functools.cache
def _make_edge_kernel():
    mesh = plsc.VectorSubcoreMesh(core_axis_name="c", subcore_axis_name="s",
                                  num_cores=2, num_subcores=16)
    return functools.partial(
        pl.kernel,
        out_type=[
            jax.ShapeDtypeStruct((2, NP, D), jnp.float32),
            jax.ShapeDtypeStruct((2, NP), jnp.float32),
        ],
        mesh=mesh,
        compiler_params=pltpu.CompilerParams(needs_layout_passes=False),
        scratch_types=[
            pltpu.VMEM_SHARED((NP, D), jnp.float32),
            pltpu.VMEM_SHARED((NP,), jnp.float32),
            pltpu.VMEM((CH,), jnp.int32),
            pltpu.VMEM((CH,), jnp.int32),
            pltpu.VMEM((NP,), jnp.float32),
            pltpu.VMEM((NP,), jnp.float32),
            pltpu.VMEM((CH,), jnp.float32),
            pltpu.VMEM((CH, D), jnp.float32),
            pltpu.VMEM((NPT,), jnp.float32),
            pltpu.SemaphoreType.DMA,
        ],
    )(_edge_body)


def _edge_body(h_hbm, asv_hbm, adv_hbm, src_hbm, dst_hbm,
               raw_hbm, den_hbm,
               acc, den_sh, src_c, dst_c, as_l, ad_l, w_c, rows,
               zbuf1, sem):
    cid = lax.axis_index("c")
    sid = lax.axis_index("s")
    wid = cid * 16 + sid

    pltpu.sync_copy(asv_hbm, as_l)
    pltpu.sync_copy(adv_hbm, ad_l)

    @pl.loop(0, NB)
    def _z(i):
        z = jnp.zeros((16,), jnp.float32)
        for g in range(D // 16):
            rows[i, pl.ds(g * 16, 16)] = z

    @pl.loop(0, NPT // 16)
    def _z1(i):
        zbuf1[pl.ds(i * 16, 16)] = jnp.zeros((16,), jnp.float32)

    nslice = pl.ds(sid * NPT, NPT)
    for p in range(NPT // NB):
        pltpu.sync_copy(rows, acc.at[pl.ds(sid * NPT + p * NB, NB)])
    pltpu.sync_copy(zbuf1, den_sh.at[nslice])
    plsc.subcore_barrier()

    ebase = wid * ET

    @pl.loop(0, NCHUNK)
    def _chunk(c):
        base = ebase + c * CH
        pltpu.sync_copy(src_hbm.at[wid * NCHUNK + c], src_c)
        pltpu.sync_copy(dst_hbm.at[wid * NCHUNK + c], dst_c)
        for g in range(CH // 16):
            s_idx = src_c[pl.ds(g * 16, 16)]
            d_idx = dst_c[pl.ds(g * 16, 16)]
            e = plsc.load_gather(as_l, [s_idx]) + plsc.load_gather(ad_l, [d_idx])
            e = jnp.maximum(e, 0.2 * e)
            w = jnp.exp(e)
            gid = base + g * 16 + lax.iota(jnp.int32, 16)
            w_c[pl.ds(g * 16, 16)] = jnp.where(gid < E, w, 0.0)
        pltpu.sync_copy(w_c, den_sh.at[dst_c], add=True)
        pltpu.async_copy(h_hbm.at[src_c], rows, sem).wait()

        @pl.loop(0, CH // 16)
        def _scale(b):
            w16 = w_c[pl.ds(b * 16, 16)]
            for i in range(16):
                s = w16[i]
                r = b * 16 + i
                for g in range(D // 16):
                    rows[r, pl.ds(g * 16, 16)] = rows[r, pl.ds(g * 16, 16)] * s

        pltpu.sync_copy(rows, acc.at[dst_c], add=True)

    plsc.subcore_barrier()

    for p in range(NPT // NB):
        rs = pl.ds(sid * NPT + p * NB, NB)
        pltpu.sync_copy(acc.at[rs], rows)
        pltpu.sync_copy(rows, raw_hbm.at[cid].at[rs])
    pltpu.sync_copy(den_sh.at[nslice], zbuf1)
    pltpu.sync_copy(zbuf1, den_hbm.at[cid].at[nslice])


def kernel(x, edge_index, W1, a_src1, a_dst1, b1, W2, a_src2, a_dst2, b2):
    src = edge_index[0].astype(jnp.int32)
    dst = edge_index[1].astype(jnp.int32)
    srcp = jnp.pad(src, (0, EP - E)).reshape(NW * NCHUNK, CH)
    dstp = jnp.pad(dst, (0, EP - E)).reshape(NW * NCHUNK, CH)
    xp = jnp.pad(x, ((0, NP - N), (0, 0)))

    edge_kernel = _make_edge_kernel()
    h, asv, adv = _dense1(xp, W1, a_src1.reshape(D, 1), a_dst1.reshape(D, 1))
    raw, den = edge_kernel(h, asv.reshape(NP), adv.reshape(NP), srcp, dstp)

    h2, asv2, adv2 = _dense2(raw, den.reshape(2, NP, 1), b1.reshape(1, D), W2,
                             a_src2.reshape(D, 1), a_dst2.reshape(D, 1))
    raw2, den2 = edge_kernel(h2, asv2.reshape(NP), adv2.reshape(NP),
                             srcp, dstp)

    return _finalize(raw2[:, :N], den2.reshape(2, NP, 1)[:, :N],
                     b2.reshape(1, D))
